# baseline (speedup 1.0000x reference)
import functools

import jax
import jax.numpy as jnp
from jax import lax
from jax.experimental import pallas as pl
from jax.experimental.pallas import tpu as pltpu
from jax.experimental.pallas import tpu_sc as plsc

_NUM_CORES = 2
_NUM_SUBCORES = 16
_NUM_WORKERS = _NUM_CORES * _NUM_SUBCORES
_LANES = 16
_IDX_CHUNK = 128


def _repack_pairs_tc(items_t):
    D, V = items_t.shape
    BLKC = 2048
    grid = (V + BLKC - 1) // BLKC

    def body(x_ref, o_ref):
        y = jnp.transpose(x_ref[...]).reshape(BLKC // 16, 2, 8, D)
        o_ref[:, :D] = y[:, 0].reshape(BLKC // 2, D)
        o_ref[:, D:] = y[:, 1].reshape(BLKC // 2, D)

    return pl.pallas_call(
        body,
        grid=(grid,),
        in_specs=[pl.BlockSpec((D, BLKC), lambda i: (0, i))],
        out_specs=pl.BlockSpec((BLKC // 2, 2 * D), lambda i: (i, 0)),
        out_shape=jax.ShapeDtypeStruct((V // 2, 2 * D), jnp.float32),
    )(items_t)


def kernel(indices, items):
    B = indices.shape[0]
    V, D = items.shape
    b_per_w = B // _NUM_WORKERS
    n_chunks = b_per_w // _IDX_CHUNK

    pairs = _repack_pairs_tc(items.T)

    mesh = plsc.VectorSubcoreMesh(core_axis_name="c", subcore_axis_name="s")

    @functools.partial(
        pl.kernel,
        mesh=mesh,
        compiler_params=pltpu.CompilerParams(needs_layout_passes=False),
        out_type=jax.ShapeDtypeStruct((D, B), jnp.float32),
        scratch_types=[
            pltpu.VMEM((b_per_w,), jnp.float32),
            pltpu.VMEM((n_chunks, _IDX_CHUNK), jnp.int32),
            pltpu.VMEM((b_per_w,), jnp.int32),
            pltpu.VMEM((b_per_w, 2 * D), jnp.float32),
            pltpu.VMEM((D, b_per_w), jnp.float32),
            pltpu.SemaphoreType.DMA,
        ],
    )
    def gather_kernel(ind_hbm, pairs_hbm, out_hbm,
                      ind_v, grp_v, sub_v, rows_v, blk_v, sem):
        wid = lax.axis_index("s") * _NUM_CORES + lax.axis_index("c")
        base = wid * b_per_w
        pltpu.sync_copy(ind_hbm.at[pl.ds(base, b_per_w)], ind_v)

        scale = jnp.float32(V)
        upper = jnp.int32(V - 1)
        copies = []
        for j in range(n_chunks):
            for i in range(_IDX_CHUNK // _LANES):
                off = j * _IDX_CHUNK + i * _LANES
                v = ind_v[pl.ds(off, _LANES)]
                v = jnp.minimum(jnp.maximum(v, jnp.float32(0.0)), jnp.float32(1.0))
                row = jnp.minimum((v * scale).astype(jnp.int32), upper)
                grp_v[j, pl.ds(i * _LANES, _LANES)] = ((row >> 4) << 3) | (row & 7)
                sub_v[pl.ds(off, _LANES)] = (row >> 3) & 1
            copies.append(pltpu.async_copy(
                pairs_hbm.at[grp_v.at[j]],
                rows_v.at[pl.ds(j * _IDX_CHUNK, _IDX_CHUNK)],
                sem,
            ))

        lane = lax.iota(jnp.int32, _LANES)
        for j in range(n_chunks):
            copies[j].wait()
            for t in range(j * _IDX_CHUNK // _LANES,
                           (j + 1) * _IDX_CHUNK // _LANES):
                i16 = t * _LANES + lane
                sub16 = sub_v[pl.ds(t * _LANES, _LANES)]
                src1 = sub16 * D

                @plsc.parallel_loop(0, _LANES, unroll=4)
                def _(k):
                    rot = (lane + k) & (_LANES - 1)
                    for cb in range(D // _LANES):
                        c = cb * _LANES + rot
                        vals = plsc.load_gather(rows_v, [i16, src1 + c])
                        plsc.store_scatter(blk_v, [c, i16], vals)

        pltpu.sync_copy(blk_v, out_hbm.at[:, pl.ds(base, b_per_w)])

    out_t = gather_kernel(indices, pairs)
    return out_t.T


# speedup vs baseline: 1.1894x; 1.1894x over previous
import functools

import jax
import jax.numpy as jnp
from jax import lax
from jax.experimental import pallas as pl
from jax.experimental.pallas import tpu as pltpu
from jax.experimental.pallas import tpu_sc as plsc

_NUM_CORES = 2
_NUM_SUBCORES = 16
_NUM_WORKERS = _NUM_CORES * _NUM_SUBCORES
_LANES = 16
_IDX_CHUNK = 128


def _repack_pairs_tc(items_t):
    D, V = items_t.shape
    BLKC = 4096
    grid = (V + BLKC - 1) // BLKC

    def body(x_ref, o_ref):
        y = jnp.transpose(x_ref[...]).reshape(BLKC // 16, 2, 8, D)
        o_ref[:, :D] = y[:, 0].reshape(BLKC // 2, D)
        o_ref[:, D:] = y[:, 1].reshape(BLKC // 2, D)

    return pl.pallas_call(
        body,
        grid=(grid,),
        in_specs=[pl.BlockSpec((D, BLKC), lambda i: (0, i))],
        out_specs=pl.BlockSpec((BLKC // 2, 2 * D), lambda i: (i, 0)),
        out_shape=jax.ShapeDtypeStruct((V // 2, 2 * D), jnp.float32),
    )(items_t)


def kernel(indices, items):
    B = indices.shape[0]
    V, D = items.shape
    b_per_w = B // _NUM_WORKERS
    n_chunks = b_per_w // _IDX_CHUNK

    pairs = _repack_pairs_tc(items.T)

    mesh = plsc.VectorSubcoreMesh(core_axis_name="c", subcore_axis_name="s")

    @functools.partial(
        pl.kernel,
        mesh=mesh,
        compiler_params=pltpu.CompilerParams(needs_layout_passes=False),
        out_type=jax.ShapeDtypeStruct((D, B), jnp.float32),
        scratch_types=[
            pltpu.VMEM((b_per_w,), jnp.float32),
            pltpu.VMEM((n_chunks, _IDX_CHUNK), jnp.int32),
            pltpu.VMEM((b_per_w,), jnp.int32),
            pltpu.VMEM((b_per_w, 2 * D), jnp.float32),
            pltpu.VMEM((D, b_per_w), jnp.float32),
            pltpu.SemaphoreType.DMA,
        ],
    )
    def gather_kernel(ind_hbm, pairs_hbm, out_hbm,
                      ind_v, grp_v, sub_v, rows_v, blk_v, sem):
        wid = lax.axis_index("s") * _NUM_CORES + lax.axis_index("c")
        base = wid * b_per_w
        pltpu.sync_copy(ind_hbm.at[pl.ds(base, b_per_w)], ind_v)

        scale = jnp.float32(V)
        upper = jnp.int32(V - 1)
        copies = []
        for j in range(n_chunks):
            for i in range(_IDX_CHUNK // _LANES):
                off = j * _IDX_CHUNK + i * _LANES
                v = ind_v[pl.ds(off, _LANES)]
                v = jnp.minimum(jnp.maximum(v, jnp.float32(0.0)), jnp.float32(1.0))
                row = jnp.minimum((v * scale).astype(jnp.int32), upper)
                grp_v[j, pl.ds(i * _LANES, _LANES)] = ((row >> 4) << 3) | (row & 7)
                sub_v[pl.ds(off, _LANES)] = (row >> 3) & 1
            copies.append(pltpu.async_copy(
                pairs_hbm.at[grp_v.at[j]],
                rows_v.at[pl.ds(j * _IDX_CHUNK, _IDX_CHUNK)],
                sem,
            ))

        lane = lax.iota(jnp.int32, _LANES)
        for j in range(n_chunks):
            copies[j].wait()
            for t in range(j * _IDX_CHUNK // _LANES,
                           (j + 1) * _IDX_CHUNK // _LANES):
                i16 = t * _LANES + lane
                sub16 = sub_v[pl.ds(t * _LANES, _LANES)]
                src1 = sub16 * D

                @plsc.parallel_loop(0, _LANES, unroll=4)
                def _(k):
                    rot = (lane + k) & (_LANES - 1)
                    for cb in range(D // _LANES):
                        c = cb * _LANES + rot
                        vals = plsc.load_gather(rows_v, [i16, src1 + c])
                        plsc.store_scatter(blk_v, [c, i16], vals)

        pltpu.sync_copy(blk_v, out_hbm.at[:, pl.ds(base, b_per_w)])

    out_t = gather_kernel(indices, pairs)
    return out_t.T


# speedup vs baseline: 1.3300x; 1.1183x over previous
import functools

import jax
import jax.numpy as jnp
from jax import lax
from jax.experimental import pallas as pl
from jax.experimental.pallas import tpu as pltpu
from jax.experimental.pallas import tpu_sc as plsc

_NUM_CORES = 2
_NUM_SUBCORES = 16
_NUM_WORKERS = _NUM_CORES * _NUM_SUBCORES
_LANES = 16
_IDX_CHUNK = 128


def _repack_pairs_tc(items_t):
    D, V = items_t.shape
    BLKC = 8192
    grid = (V + BLKC - 1) // BLKC

    def body(x_ref, o_ref):
        y = jnp.transpose(x_ref[...]).reshape(BLKC // 16, 2, 8, D)
        o_ref[:, :D] = y[:, 0].reshape(BLKC // 2, D)
        o_ref[:, D:] = y[:, 1].reshape(BLKC // 2, D)

    return pl.pallas_call(
        body,
        grid=(grid,),
        in_specs=[pl.BlockSpec((D, BLKC), lambda i: (0, i))],
        out_specs=pl.BlockSpec((BLKC // 2, 2 * D), lambda i: (i, 0)),
        out_shape=jax.ShapeDtypeStruct((V // 2, 2 * D), jnp.float32),
    )(items_t)


def kernel(indices, items):
    B = indices.shape[0]
    V, D = items.shape
    b_per_w = B // _NUM_WORKERS
    n_chunks = b_per_w // _IDX_CHUNK

    pairs = _repack_pairs_tc(items.T)

    mesh = plsc.VectorSubcoreMesh(core_axis_name="c", subcore_axis_name="s")

    @functools.partial(
        pl.kernel,
        mesh=mesh,
        compiler_params=pltpu.CompilerParams(needs_layout_passes=False),
        out_type=jax.ShapeDtypeStruct((D, B), jnp.float32),
        scratch_types=[
            pltpu.VMEM((b_per_w,), jnp.float32),
            pltpu.VMEM((n_chunks, _IDX_CHUNK), jnp.int32),
            pltpu.VMEM((b_per_w,), jnp.int32),
            pltpu.VMEM((b_per_w, 2 * D), jnp.float32),
            pltpu.VMEM((D, b_per_w), jnp.float32),
            pltpu.SemaphoreType.DMA,
        ],
    )
    def gather_kernel(ind_hbm, pairs_hbm, out_hbm,
                      ind_v, grp_v, sub_v, rows_v, blk_v, sem):
        wid = lax.axis_index("s") * _NUM_CORES + lax.axis_index("c")
        base = wid * b_per_w
        pltpu.sync_copy(ind_hbm.at[pl.ds(base, b_per_w)], ind_v)

        scale = jnp.float32(V)
        upper = jnp.int32(V - 1)
        copies = []
        for j in range(n_chunks):
            for i in range(_IDX_CHUNK // _LANES):
                off = j * _IDX_CHUNK + i * _LANES
                v = ind_v[pl.ds(off, _LANES)]
                v = jnp.minimum(jnp.maximum(v, jnp.float32(0.0)), jnp.float32(1.0))
                row = jnp.minimum((v * scale).astype(jnp.int32), upper)
                grp_v[j, pl.ds(i * _LANES, _LANES)] = ((row >> 4) << 3) | (row & 7)
                sub_v[pl.ds(off, _LANES)] = (row >> 3) & 1
            copies.append(pltpu.async_copy(
                pairs_hbm.at[grp_v.at[j]],
                rows_v.at[pl.ds(j * _IDX_CHUNK, _IDX_CHUNK)],
                sem,
            ))

        lane = lax.iota(jnp.int32, _LANES)
        for j in range(n_chunks):
            copies[j].wait()
            for t in range(j * _IDX_CHUNK // _LANES,
                           (j + 1) * _IDX_CHUNK // _LANES):
                i16 = t * _LANES + lane
                sub16 = sub_v[pl.ds(t * _LANES, _LANES)]
                src1 = sub16 * D

                @plsc.parallel_loop(0, _LANES, unroll=4)
                def _(k):
                    rot = (lane + k) & (_LANES - 1)
                    for cb in range(D // _LANES):
                        c = cb * _LANES + rot
                        vals = plsc.load_gather(rows_v, [i16, src1 + c])
                        plsc.store_scatter(blk_v, [c, i16], vals)

        pltpu.sync_copy(blk_v, out_hbm.at[:, pl.ds(base, b_per_w)])

    out_t = gather_kernel(indices, pairs)
    return out_t.T


# speedup vs baseline: 1.3545x; 1.0184x over previous
import functools

import jax
import jax.numpy as jnp
from jax import lax
from jax.experimental import pallas as pl
from jax.experimental.pallas import tpu as pltpu
from jax.experimental.pallas import tpu_sc as plsc

_NUM_CORES = 2
_NUM_SUBCORES = 16
_NUM_WORKERS = _NUM_CORES * _NUM_SUBCORES
_LANES = 16
_IDX_CHUNK = 128


def _repack_pairs_tc(items_t):
    D, V = items_t.shape
    BLKC = 16384
    grid = (V + BLKC - 1) // BLKC

    def body(x_ref, o_ref):
        y = jnp.transpose(x_ref[...]).reshape(BLKC // 16, 2, 8, D)
        o_ref[:, :D] = y[:, 0].reshape(BLKC // 2, D)
        o_ref[:, D:] = y[:, 1].reshape(BLKC // 2, D)

    return pl.pallas_call(
        body,
        grid=(grid,),
        in_specs=[pl.BlockSpec((D, BLKC), lambda i: (0, i))],
        out_specs=pl.BlockSpec((BLKC // 2, 2 * D), lambda i: (i, 0)),
        out_shape=jax.ShapeDtypeStruct((V // 2, 2 * D), jnp.float32),
    )(items_t)


def kernel(indices, items):
    B = indices.shape[0]
    V, D = items.shape
    b_per_w = B // _NUM_WORKERS
    n_chunks = b_per_w // _IDX_CHUNK

    pairs = _repack_pairs_tc(items.T)

    mesh = plsc.VectorSubcoreMesh(core_axis_name="c", subcore_axis_name="s")

    @functools.partial(
        pl.kernel,
        mesh=mesh,
        compiler_params=pltpu.CompilerParams(needs_layout_passes=False),
        out_type=jax.ShapeDtypeStruct((D, B), jnp.float32),
        scratch_types=[
            pltpu.VMEM((b_per_w,), jnp.float32),
            pltpu.VMEM((n_chunks, _IDX_CHUNK), jnp.int32),
            pltpu.VMEM((b_per_w,), jnp.int32),
            pltpu.VMEM((b_per_w, 2 * D), jnp.float32),
            pltpu.VMEM((D, b_per_w), jnp.float32),
            pltpu.SemaphoreType.DMA,
        ],
    )
    def gather_kernel(ind_hbm, pairs_hbm, out_hbm,
                      ind_v, grp_v, sub_v, rows_v, blk_v, sem):
        wid = lax.axis_index("s") * _NUM_CORES + lax.axis_index("c")
        base = wid * b_per_w
        pltpu.sync_copy(ind_hbm.at[pl.ds(base, b_per_w)], ind_v)

        scale = jnp.float32(V)
        upper = jnp.int32(V - 1)
        copies = []
        for j in range(n_chunks):
            for i in range(_IDX_CHUNK // _LANES):
                off = j * _IDX_CHUNK + i * _LANES
                v = ind_v[pl.ds(off, _LANES)]
                v = jnp.minimum(jnp.maximum(v, jnp.float32(0.0)), jnp.float32(1.0))
                row = jnp.minimum((v * scale).astype(jnp.int32), upper)
                grp_v[j, pl.ds(i * _LANES, _LANES)] = ((row >> 4) << 3) | (row & 7)
                sub_v[pl.ds(off, _LANES)] = (row >> 3) & 1
            copies.append(pltpu.async_copy(
                pairs_hbm.at[grp_v.at[j]],
                rows_v.at[pl.ds(j * _IDX_CHUNK, _IDX_CHUNK)],
                sem,
            ))

        lane = lax.iota(jnp.int32, _LANES)
        for j in range(n_chunks):
            copies[j].wait()
            for t in range(j * _IDX_CHUNK // _LANES,
                           (j + 1) * _IDX_CHUNK // _LANES):
                i16 = t * _LANES + lane
                sub16 = sub_v[pl.ds(t * _LANES, _LANES)]
                src1 = sub16 * D

                @plsc.parallel_loop(0, _LANES, unroll=4)
                def _(k):
                    rot = (lane + k) & (_LANES - 1)
                    for cb in range(D // _LANES):
                        c = cb * _LANES + rot
                        vals = plsc.load_gather(rows_v, [i16, src1 + c])
                        plsc.store_scatter(blk_v, [c, i16], vals)

        pltpu.sync_copy(blk_v, out_hbm.at[:, pl.ds(base, b_per_w)])

    out_t = gather_kernel(indices, pairs)
    return out_t.T
